# Initial kernel scaffold; baseline (speedup 1.0000x reference)
import jax
import jax.numpy as jnp
from jax.experimental import pallas as pl


def kernel(layer, batch, times, x, ptr, idx, ts, W1, b1, W2, b2):
    raise NotImplementedError("write your pallas kernel here")


# speedup vs baseline: 11.7134x; 11.7134x over previous
import functools

import jax
import jax.numpy as jnp
from jax import lax
from jax.experimental import pallas as pl
from jax.experimental.pallas import tpu as pltpu
from jax.experimental.pallas import tpu_sc as plsc

N = 10000
D = 128
B = 1024
NW = 32
QPW = B // NW
G = D // 16
STG = 136
STGB = 152
PTRB = 10016


def _f32(v):
    return jnp.float32(v)


def _sc_body(mode_h, batch_h, times_h, x_h, ptr_h, idx_h, ts_h,
             w1_h, b1_h, w2_h, b2_h, out_h,
             ptr_v, w1_v, w2_v, b1_v, b2_v, bt_v, tm_v, md_v,
             seg1i, seg1t, seg2i, seg2t, g16, rows16, bself_v, selfr_v,
             acc_v, s2_v, out_v, sem):
    wid = lax.axis_index("s") * 2 + lax.axis_index("c")
    base = wid * QPW

    def sload(ref, i):
        return ref[pl.ds(i, 16)][0]

    pltpu.sync_copy(ptr_h, ptr_v)
    pltpu.sync_copy(w1_h, w1_v)
    pltpu.sync_copy(w2_h, w2_v)
    pltpu.sync_copy(b1_h, b1_v)
    pltpu.sync_copy(b2_h, b2_v)
    pltpu.sync_copy(mode_h, md_v)
    pltpu.sync_copy(batch_h.at[pl.ds(base, QPW)], bt_v.at[pl.ds(0, QPW)])
    pltpu.sync_copy(times_h.at[pl.ds(base, QPW)], tm_v.at[pl.ds(0, QPW)])
    bt_v[pl.ds(QPW, 16)] = jnp.zeros((16,), jnp.int32)
    pltpu.async_copy(x_h.at[bt_v], bself_v, sem).wait()

    def level1(j2, t2, self_ref, srow):
        pv = ptr_v[pl.ds(j2, 16)]
        lo = pv[0]
        hi = pv[1]
        n = hi - lo
        for g in range(G):
            sl = pl.ds(16 * g, 16)
            acc_v[sl] = self_ref[srow, sl]
        nch = (n + 127) // 128

        def chunk(ch, cnt):
            start = lo + ch * 128
            s8 = (start // 8) * 8
            off = start - s8
            m = jnp.minimum(128, n - ch * 128)
            pltpu.sync_copy(idx_h.at[pl.ds(s8, STG)], seg1i.at[pl.ds(0, STG)])
            pltpu.sync_copy(ts_h.at[pl.ds(s8, STG)], seg1t.at[pl.ds(0, STG)])
            nsub = (m + 15) // 16

            def sub(s, cnt):
                g16[:] = seg1i[pl.ds(off + s * 16, 16)]
                pltpu.async_copy(x_h.at[g16], rows16, sem).wait()
                mm = jnp.minimum(16, m - s * 16)

                def row(r, cnt):
                    te = sload(seg1t, off + s * 16 + r)
                    ok = te <= t2
                    f = jnp.where(ok, _f32(1.0), _f32(0.0))
                    for g in range(G):
                        sl = pl.ds(16 * g, 16)
                        acc_v[sl] = acc_v[sl] + f * rows16[r, sl]
                    return cnt + ok.astype(jnp.int32)

                return lax.fori_loop(0, mm, row, cnt)

            return lax.fori_loop(0, nsub, sub, cnt)

        cnt = lax.fori_loop(0, nch, chunk, jnp.int32(0))
        den = jnp.full((16,), _f32(1.0)) * (cnt.astype(jnp.float32) + _f32(1.0))
        inv = jnp.full((16,), _f32(1.0)) / den
        for g in range(G):
            sl = pl.ds(16 * g, 16)
            acc_v[sl] = acc_v[sl] * inv

    def matvec(w_v, b_v):
        z = tuple(b_v[pl.ds(16 * g, 16)] for g in range(G))

        def d_step(d, z):
            s = sload(acc_v, d)
            return tuple(z[g] + s * w_v[d, pl.ds(16 * g, 16)] for g in range(G))

        z = lax.fori_loop(0, D, d_step, z)
        return tuple(jnp.maximum(z[g], _f32(0.0)) for g in range(G))

    def per_query(q, _):
        j = sload(bt_v, q)
        t = sload(tm_v, q)
        mode = sload(md_v, 0)

        @pl.when(mode == 0)
        def _():
            for g in range(G):
                sl = pl.ds(16 * g, 16)
                out_v[q, sl] = bself_v[q, sl]

        @pl.when(mode >= 1)
        def _():
            level1(j, t, bself_v, q)
            h1b = matvec(w1_v, b1_v)

            @pl.when(mode == 1)
            def _():
                for g in range(G):
                    out_v[q, pl.ds(16 * g, 16)] = h1b[g]

            @pl.when(mode == 2)
            def _():
                for g in range(G):
                    s2_v[pl.ds(16 * g, 16)] = jnp.zeros((16,), jnp.float32)
                pv = ptr_v[pl.ds(j, 16)]
                lo = pv[0]
                hi = pv[1]
                n = hi - lo
                nch = (n + 127) // 128

                def chunk(ch, c2):
                    start = lo + ch * 128
                    s8 = (start // 8) * 8
                    off = start - s8
                    m = jnp.minimum(128, n - ch * 128)
                    pltpu.sync_copy(idx_h.at[pl.ds(s8, STG)],
                                    seg2i.at[pl.ds(0, STG)])
                    pltpu.sync_copy(ts_h.at[pl.ds(s8, STG)],
                                    seg2t.at[pl.ds(0, STG)])
                    for k in range(8):
                        @pl.when(k * 16 < m)
                        def _():
                            g16[:] = seg2i[pl.ds(off + k * 16, 16)]
                            pltpu.async_copy(
                                x_h.at[g16],
                                selfr_v.at[pl.ds(k * 16, 16)], sem).wait()

                    def row(r, c2):
                        te = sload(seg2t, off + r)

                        def do_edge(c2):
                            j2 = sload(seg2i, off + r)
                            level1(j2, te, selfr_v, r)
                            h = matvec(w1_v, b1_v)
                            for g in range(G):
                                sl = pl.ds(16 * g, 16)
                                s2_v[sl] = s2_v[sl] + h[g]
                            return c2 + 1

                        return lax.cond(te <= t, do_edge, lambda c: c, c2)

                    return lax.fori_loop(0, m, row, c2)

                c2 = lax.fori_loop(0, nch, chunk, jnp.int32(0))
                den2 = (jnp.full((16,), _f32(1.0))
                        * (c2.astype(jnp.float32) + _f32(1.0)))
                inv2 = jnp.full((16,), _f32(1.0)) / den2
                for g in range(G):
                    sl = pl.ds(16 * g, 16)
                    acc_v[sl] = (s2_v[sl] + h1b[g]) * inv2
                z2 = matvec(w2_v, b2_v)
                for g in range(G):
                    out_v[q, pl.ds(16 * g, 16)] = z2[g]

        return 0

    lax.fori_loop(0, QPW, per_query, 0)
    pltpu.sync_copy(out_v, out_h.at[pl.ds(base, QPW)])


@jax.jit
def _flow_gnn_sc(mode, batch, times, x, ptr, idx, ts, w1t, b1, w2t, b2):
    mesh = plsc.VectorSubcoreMesh(core_axis_name="c", subcore_axis_name="s")
    kfn = functools.partial(
        pl.kernel,
        mesh=mesh,
        out_type=jax.ShapeDtypeStruct((B, D), jnp.float32),
        scratch_types=[
            pltpu.VMEM((PTRB,), jnp.int32),
            pltpu.VMEM((D, D), jnp.float32),
            pltpu.VMEM((D, D), jnp.float32),
            pltpu.VMEM((D,), jnp.float32),
            pltpu.VMEM((D,), jnp.float32),
            pltpu.VMEM((QPW + 16,), jnp.int32),
            pltpu.VMEM((QPW + 16,), jnp.int32),
            pltpu.VMEM((16,), jnp.int32),
            pltpu.VMEM((STGB,), jnp.int32),
            pltpu.VMEM((STGB,), jnp.int32),
            pltpu.VMEM((STGB,), jnp.int32),
            pltpu.VMEM((STGB,), jnp.int32),
            pltpu.VMEM((16,), jnp.int32),
            pltpu.VMEM((16, D), jnp.float32),
            pltpu.VMEM((QPW + 16, D), jnp.float32),
            pltpu.VMEM((128, D), jnp.float32),
            pltpu.VMEM((D + 16,), jnp.float32),
            pltpu.VMEM((D,), jnp.float32),
            pltpu.VMEM((QPW, D), jnp.float32),
            pltpu.SemaphoreType.DMA,
        ],
    )(_sc_body)
    return kfn(mode, batch, times, x, ptr, idx, ts, w1t, b1, w2t, b2)


def kernel(layer, batch, times, x, ptr, idx, ts, W1, b1, W2, b2):
    batch = jnp.asarray(batch).astype(jnp.int32)
    times = jnp.asarray(times).astype(jnp.int32)
    ptr = jnp.asarray(ptr).astype(jnp.int32)
    idx = jnp.asarray(idx).astype(jnp.int32)
    ts = jnp.asarray(ts).astype(jnp.int32)
    idx_p = jnp.concatenate([idx, jnp.zeros((STG + 8,), jnp.int32)])
    ts_p = jnp.concatenate([ts, jnp.full((STG + 8,), jnp.iinfo(jnp.int32).max,
                                         jnp.int32)])
    ptr_p = jnp.concatenate(
        [ptr, jnp.full((PTRB - (N + 1),), ptr[-1], jnp.int32)])
    mode = jnp.full((16,), jnp.clip(jnp.asarray(layer, jnp.int32), 0, 2),
                    jnp.int32)
    w1t = jnp.asarray(W1).T.copy()
    w2t = jnp.asarray(W2).T.copy()
    return _flow_gnn_sc(mode, batch, times, x, ptr_p, idx_p, ts_p,
                        w1t, b1, w2t, b2)
